# baseline (speedup 1.0000x reference)
import functools
import math

import jax
import jax.numpy as jnp
from jax import lax
from jax.experimental import pallas as pl
from jax.experimental.pallas import tpu as pltpu
from jax.experimental.pallas import tpu_sc as plsc

_N_FIELDS = 26
_VOCAB = 100000
_EMB_DIM = 32
_D_MODEL = 1024
_BATCH = 4096
_SUM_EMB = _N_FIELDS * _EMB_DIM
_SCALE = math.sqrt(_EMB_DIM)

_NC = 2
_NS = 16
_NW = _NC * _NS
_L = 16

_BPW = _BATCH // _NW
_RPW = _BPW * _N_FIELDS
_CHUNK = _BPW
_NGB = 4
_NWB = 2


def _gather_body(x_hbm, tab_hbm, out_hbm, xv, idxv, colv, padv, comp,
                 sg0, sg1, sg2, sg3, sw0, sw1):
    gsems = (sg0, sg1, sg2, sg3)
    wsems = (sw0, sw1)
    wid = lax.axis_index("s") * _NC + lax.axis_index("c")
    base = wid * _RPW
    pltpu.sync_copy(x_hbm.at[pl.ds(base, _RPW)], xv)

    lanes = lax.iota(jnp.int32, _L)

    def idx_body(c, carry):
        for g in range(_CHUNK // _L):
            j = g * _L + lanes
            xval = plsc.load_gather(xv, [j * _N_FIELDS + c])
            q = xval + c * _VOCAB
            idxv[c, pl.ds(g * _L, _L)] = lax.shift_right_logical(q, 2)
            colv[c, pl.ds(g * _L, _L)] = lax.shift_left(
                lax.bitwise_and(q, 3), 5)
        return carry

    lax.fori_loop(0, _N_FIELDS, idx_body, 0)

    def start_gather(c):
        pltpu.make_async_copy(
            tab_hbm.at[idxv.at[c]], padv.at[c % _NGB], gsems[c % _NGB]
        ).start()

    def wait_gather(c):
        pltpu.make_async_copy(
            tab_hbm.at[idxv.at[c]], padv.at[c % _NGB], gsems[c % _NGB]
        ).wait()

    def out_slice(c):
        return out_hbm.at[c, pl.ds(wid * _BPW, _BPW), :]

    def start_write(c):
        pltpu.make_async_copy(comp.at[c % _NWB], out_slice(c), wsems[c % _NWB]).start()

    def wait_write(c):
        pltpu.make_async_copy(comp.at[c % _NWB], out_slice(c), wsems[c % _NWB]).wait()

    for c in range(_NGB):
        start_gather(c)

    for c in range(_N_FIELDS):
        wait_gather(c)
        if c >= _NWB:
            wait_write(c - _NWB)
        slot = c % _NGB
        wslot = c % _NWB
        slotv = jnp.full((_L,), slot, jnp.int32)
        wslotv = jnp.full((_L,), wslot, jnp.int32)

        def comp_body(g, carry, slotv=slotv, wslotv=wslotv, c=c):
            r16 = g * _L + lanes
            co16 = colv[c, pl.ds(g * _L, _L)]
            for e in range(_EMB_DIM):
                v = plsc.load_gather(padv, [slotv, r16, co16 + e])
                plsc.store_scatter(
                    comp, [wslotv, r16, jnp.full((_L,), e, jnp.int32)], v)
            return carry

        lax.fori_loop(0, _CHUNK // _L, comp_body, 0)
        start_write(c)
        if c + _NGB < _N_FIELDS:
            start_gather(c + _NGB)

    wait_write(_N_FIELDS - _NWB)
    wait_write(_N_FIELDS - 1)


@functools.cache
def _make_gather():
    return pl.kernel(
        _gather_body,
        out_type=jax.ShapeDtypeStruct((_N_FIELDS, _BATCH, _EMB_DIM), jnp.float32),
        mesh=plsc.VectorSubcoreMesh(core_axis_name="c", subcore_axis_name="s"),
        scratch_types=[
            pltpu.VMEM((_RPW,), jnp.int32),
            pltpu.VMEM((_N_FIELDS, _CHUNK), jnp.int32),
            pltpu.VMEM((_N_FIELDS, _CHUNK), jnp.int32),
            pltpu.VMEM((_NGB, _CHUNK, 128), jnp.float32),
            pltpu.VMEM((_NWB, _CHUNK, _EMB_DIM), jnp.float32),
            pltpu.SemaphoreType.DMA,
            pltpu.SemaphoreType.DMA,
            pltpu.SemaphoreType.DMA,
            pltpu.SemaphoreType.DMA,
            pltpu.SemaphoreType.DMA,
            pltpu.SemaphoreType.DMA,
        ],
        compiler_params=pltpu.CompilerParams(needs_layout_passes=False),
    )


def _proj_body(e_ref, w_ref, b_ref, o_ref):
    f = pl.program_id(1)

    @pl.when(f == 0)
    def _():
        o_ref[...] = jnp.broadcast_to(b_ref[...], o_ref.shape)

    o_ref[...] += (
        jnp.dot(e_ref[0], w_ref[0], preferred_element_type=jnp.float32)
        * _SCALE
    )


_M_TILE = 512

_proj = pl.pallas_call(
    _proj_body,
    grid=(_BATCH // _M_TILE, _N_FIELDS),
    in_specs=[
        pl.BlockSpec((1, _M_TILE, _EMB_DIM), lambda i, f: (f, i, 0)),
        pl.BlockSpec((1, _EMB_DIM, _D_MODEL), lambda i, f: (f, 0, 0)),
        pl.BlockSpec((1, _D_MODEL), lambda i, f: (0, 0)),
    ],
    out_specs=pl.BlockSpec((_M_TILE, _D_MODEL), lambda i, f: (i, 0)),
    out_shape=jax.ShapeDtypeStruct((_BATCH, _D_MODEL), jnp.float32),
    compiler_params=pltpu.CompilerParams(
        dimension_semantics=("parallel", "arbitrary")),
)


def kernel(x, tables, W, b):
    x_flat = x.reshape(_BATCH * _N_FIELDS)
    tab128 = tables.reshape(_N_FIELDS * _VOCAB * _EMB_DIM // 128, 128)
    embs = _make_gather()(x_flat, tab128)
    w3 = W.reshape(_N_FIELDS, _EMB_DIM, _D_MODEL)
    return _proj(embs, w3, b.reshape(1, _D_MODEL))


# speedup vs baseline: 1.2064x; 1.2064x over previous
import functools
import math

import jax
import jax.numpy as jnp
from jax import lax
from jax.experimental import pallas as pl
from jax.experimental.pallas import tpu as pltpu
from jax.experimental.pallas import tpu_sc as plsc

_N_FIELDS = 26
_VOCAB = 100000
_EMB_DIM = 32
_D_MODEL = 1024
_BATCH = 4096
_SUM_EMB = _N_FIELDS * _EMB_DIM
_SCALE = math.sqrt(_EMB_DIM)

_NC = 2
_NS = 16
_NW = _NC * _NS
_BPW = _BATCH // _NW


def _gather_body(xt_hbm, tab_hbm, out_hbm, idxv, rows, gsem, wsem):
    wid = lax.axis_index("s") * _NC + lax.axis_index("c")
    base = wid * _BPW
    pltpu.sync_copy(xt_hbm.at[:, pl.ds(base, _BPW)], idxv)

    def gather(c):
        return pltpu.make_async_copy(
            tab_hbm.at[c].at[idxv.at[c]], rows.at[c], gsem)

    def write(c):
        return pltpu.make_async_copy(
            rows.at[c],
            out_hbm.at[pl.ds(base, _BPW), pl.ds(c * _EMB_DIM, _EMB_DIM)],
            wsem)

    for c in range(_N_FIELDS):
        gather(c).start()
    for c in range(_N_FIELDS):
        gather(c).wait()
    for c in range(_N_FIELDS):
        write(c).start()
    for c in range(_N_FIELDS):
        write(c).wait()


@functools.cache
def _make_gather():
    return pl.kernel(
        _gather_body,
        out_type=jax.ShapeDtypeStruct((_BATCH, _SUM_EMB), jnp.float32),
        mesh=plsc.VectorSubcoreMesh(core_axis_name="c", subcore_axis_name="s"),
        scratch_types=[
            pltpu.VMEM((_N_FIELDS, _BPW), jnp.int32),
            pltpu.VMEM((_N_FIELDS, _BPW, _EMB_DIM), jnp.float32),
            pltpu.SemaphoreType.DMA,
            pltpu.SemaphoreType.DMA,
        ],
        compiler_params=pltpu.CompilerParams(use_tc_tiling_on_sc=False),
    )


def _proj_body(e_ref, w_ref, b_ref, o_ref):
    acc = jnp.dot(e_ref[...], w_ref[...], preferred_element_type=jnp.float32)
    o_ref[...] = acc * _SCALE + b_ref[...]


_M_TILE = 512

_proj = pl.pallas_call(
    _proj_body,
    grid=(_BATCH // _M_TILE,),
    in_specs=[
        pl.BlockSpec((_M_TILE, _SUM_EMB), lambda i: (i, 0)),
        pl.BlockSpec((_SUM_EMB, _D_MODEL), lambda i: (0, 0)),
        pl.BlockSpec((1, _D_MODEL), lambda i: (0, 0)),
    ],
    out_specs=pl.BlockSpec((_M_TILE, _D_MODEL), lambda i: (i, 0)),
    out_shape=jax.ShapeDtypeStruct((_BATCH, _D_MODEL), jnp.float32),
)


def kernel(x, tables, W, b):
    embs = _make_gather()(x.T, tables)
    return _proj(embs, W, b.reshape(1, _D_MODEL))


# speedup vs baseline: 1.9211x; 1.5924x over previous
import functools
import math

import jax
import jax.numpy as jnp
from jax import lax
from jax.experimental import pallas as pl
from jax.experimental.pallas import tpu as pltpu
from jax.experimental.pallas import tpu_sc as plsc

_N_FIELDS = 26
_VOCAB = 100000
_EMB_DIM = 32
_D_MODEL = 1024
_BATCH = 4096
_SUM_EMB = _N_FIELDS * _EMB_DIM
_SCALE = math.sqrt(_EMB_DIM)

_NC = 2
_NS = 16
_NW = _NC * _NS
_L = 16
_BPW = _BATCH // _NW
_CPR = _VOCAB // 8


def _gather_body(xt_hbm, tab_hbm, out_hbm, xall, idxv, offv, chunks, strip,
                 gsem, wsem):
    wid = lax.axis_index("s") * _NC + lax.axis_index("c")
    base = wid * _BPW
    pltpu.sync_copy(xt_hbm.at[:, pl.ds(base, _BPW)], xall)

    lanes = lax.iota(jnp.int32, _L)

    def field_body(f, carry):
        def build(e, c2):
            k = f * _EMB_DIM + e
            for g in range(_BPW // _L):
                xv = xall[f, pl.ds(g * _L, _L)]
                idxv[e, pl.ds(g * _L, _L)] = (
                    lax.shift_right_logical(xv, 3) + k * _CPR)
            return c2

        lax.fori_loop(0, _EMB_DIM, build, 0)
        for g in range(_BPW // _L):
            offv[g, :] = lax.bitwise_and(xall[f, pl.ds(g * _L, _L)], 7)

        for e in range(_EMB_DIM):
            pltpu.make_async_copy(
                tab_hbm.at[idxv.at[e]], chunks.at[e], gsem).start()
        for e in range(_EMB_DIM):
            pltpu.make_async_copy(
                tab_hbm.at[idxv.at[e]], chunks.at[e], gsem).wait()

        @pl.when(f > 0)
        def _():
            pltpu.make_async_copy(
                strip,
                out_hbm.at[pl.ds((f - 1) * _EMB_DIM, _EMB_DIM),
                           pl.ds(base, _BPW)],
                wsem,
            ).wait()

        def extract(e, c2):
            ev = jnp.zeros((_L,), jnp.int32) + e
            for g in range(_BPW // _L):
                b16 = g * _L + lanes
                v = plsc.load_gather(chunks, [ev, b16, offv[g, :]])
                strip[e, pl.ds(g * _L, _L)] = v
            return c2

        lax.fori_loop(0, _EMB_DIM, extract, 0)

        pltpu.make_async_copy(
            strip,
            out_hbm.at[pl.ds(f * _EMB_DIM, _EMB_DIM), pl.ds(base, _BPW)],
            wsem,
        ).start()
        return carry

    lax.fori_loop(0, _N_FIELDS, field_body, 0)

    pltpu.make_async_copy(
        strip,
        out_hbm.at[pl.ds((_N_FIELDS - 1) * _EMB_DIM, _EMB_DIM),
                   pl.ds(base, _BPW)],
        wsem,
    ).wait()


@functools.cache
def _make_gather():
    return pl.kernel(
        _gather_body,
        out_type=jax.ShapeDtypeStruct((_SUM_EMB, _BATCH), jnp.float32),
        mesh=plsc.VectorSubcoreMesh(core_axis_name="c", subcore_axis_name="s"),
        scratch_types=[
            pltpu.VMEM((_N_FIELDS, _BPW), jnp.int32),
            pltpu.VMEM((_EMB_DIM, _BPW), jnp.int32),
            pltpu.VMEM((_BPW // _L, _L), jnp.int32),
            pltpu.VMEM((_EMB_DIM, _BPW, 8), jnp.float32),
            pltpu.VMEM((_EMB_DIM, _BPW), jnp.float32),
            pltpu.SemaphoreType.DMA,
            pltpu.SemaphoreType.DMA,
        ],
        compiler_params=pltpu.CompilerParams(
            use_tc_tiling_on_sc=False, needs_layout_passes=False),
    )


def _proj_body(e_ref, w_ref, b_ref, o_ref):
    acc = jax.lax.dot_general(
        e_ref[...], w_ref[...],
        dimension_numbers=(((0,), (0,)), ((), ())),
        preferred_element_type=jnp.float32)
    o_ref[...] = acc * _SCALE + b_ref[...]


_M_TILE = 512

_proj = pl.pallas_call(
    _proj_body,
    grid=(_BATCH // _M_TILE,),
    in_specs=[
        pl.BlockSpec((_SUM_EMB, _M_TILE), lambda i: (0, i)),
        pl.BlockSpec((_SUM_EMB, _D_MODEL), lambda i: (0, 0)),
        pl.BlockSpec((1, _D_MODEL), lambda i: (0, 0)),
    ],
    out_specs=pl.BlockSpec((_M_TILE, _D_MODEL), lambda i: (i, 0)),
    out_shape=jax.ShapeDtypeStruct((_BATCH, _D_MODEL), jnp.float32),
)


def kernel(x, tables, W, b):
    tabt = tables.transpose(0, 2, 1).reshape(_SUM_EMB, _VOCAB)
    tabc = tabt.reshape(_SUM_EMB * _CPR, 8)
    embt = _make_gather()(x.T, tabc)
    return _proj(embt, W, b.reshape(1, _D_MODEL))
